# baseline (speedup 1.0000x reference)
import jax
import jax.numpy as jnp
from jax.experimental import pallas as pl
from jax.experimental.pallas import tpu as pltpu

_NUM_CLASSES = 400
_LABEL_SMOOTH = 0.1
_ALPHA = 1.0
_IGNORE_CLS = -1
_B = 16


def _mix_x_body(partner_ref, lam_ref, xa_ref, xb1_ref, xb2_ref, o_ref):
    i = pl.program_id(0)
    lam = lam_ref[i]
    ht = xa_ref.shape[2] // 2
    o_ref[:, :, :ht] = xa_ref[:, :, :ht] * lam + xb1_ref[...] * (1.0 - lam)
    o_ref[:, :, ht:] = xa_ref[:, :, ht:] * lam + xb2_ref[...] * (1.0 - lam)


def _mix_x(x, partner, lam_rows):
    b, c, t, h, w = x.shape
    grid_spec = pltpu.PrefetchScalarGridSpec(
        num_scalar_prefetch=2,
        grid=(b, c),
        in_specs=[
            pl.BlockSpec((1, 1, t, h, w), lambda i, j, p, l: (i, j, 0, 0, 0)),
            pl.BlockSpec((1, 1, t // 2, h, w), lambda i, j, p, l: (p[i], j, 0, 0, 0)),
            pl.BlockSpec((1, 1, t // 2, h, w), lambda i, j, p, l: (p[i], j, 1, 0, 0)),
        ],
        out_specs=pl.BlockSpec((1, 1, t, h, w), lambda i, j, p, l: (i, j, 0, 0, 0)),
    )
    return pl.pallas_call(
        _mix_x_body,
        grid_spec=grid_spec,
        out_shape=jax.ShapeDtypeStruct(x.shape, jnp.float32),
        compiler_params=pltpu.CompilerParams(
            dimension_semantics=("arbitrary", "arbitrary"),
        ),
    )(partner, lam_rows, x, x, x)


def _one_hot_smooth(t):
    nt = _LABEL_SMOOTH / _NUM_CLASSES
    tv = 1.0 - _LABEL_SMOOTH + nt
    hot = jax.nn.one_hot(jnp.squeeze(t, axis=-1), _NUM_CLASSES, dtype=jnp.float32)
    return jnp.where(hot > 0.5, jnp.float32(tv), jnp.float32(nt))


def kernel(x_video_rgb, labels_action, labels_subclips_action):
    ts = jnp.squeeze(labels_subclips_action, axis=-1)
    mask = jnp.all(ts != _IGNORE_CLS, axis=-1)
    k = jnp.sum(mask)
    no_mix = k <= 1
    order = jnp.argsort(jnp.logical_not(mask), stable=True)
    rank = jnp.cumsum(mask) - 1
    partner = order[jnp.clip(k - 1 - rank, 0, _B - 1)].astype(jnp.int32)
    lam = jax.random.beta(jax.random.key(1), _ALPHA, _ALPHA)
    mix_on = mask & jnp.logical_not(no_mix)
    lam_rows = jnp.where(mix_on, lam, 1.0).astype(jnp.float32)

    x_out = _mix_x(x_video_rgb, partner, lam_rows)

    labels_out = _one_hot_smooth(labels_action)
    subclips_ignore_index = labels_subclips_action == _IGNORE_CLS
    val_tmp = jnp.where(subclips_ignore_index, 0, labels_subclips_action)
    labels_subclips_out = _one_hot_smooth(val_tmp)

    lam_c = lam_rows[:, None]
    labels_out = lam_c * labels_out + (1.0 - lam_c) * labels_out[partner]
    lam_s = lam_rows[:, None, None]
    labels_subclips_out = (
        lam_s * labels_subclips_out + (1.0 - lam_s) * labels_subclips_out[partner]
    )
    return (x_out, labels_out, labels_subclips_out, subclips_ignore_index)


# speedup vs baseline: 1.2444x; 1.2444x over previous
import jax
import jax.numpy as jnp
from jax.experimental import pallas as pl
from jax.experimental.pallas import tpu as pltpu

_NUM_CLASSES = 400
_LABEL_SMOOTH = 0.1
_ALPHA = 1.0
_IGNORE_CLS = -1
_B = 16


def _mix_x_body(partner_ref, lam_ref, x_ref, o_ref):
    for i in range(_B):
        lam = lam_ref[i]
        p = partner_ref[i]
        xi = x_ref[pl.ds(i, 1)]
        xp = x_ref[pl.ds(p, 1)]
        o_ref[pl.ds(i, 1)] = xi * lam + xp * (1.0 - lam)


def _mix_x(x, partner, lam_rows, tchunk):
    b, c, t, h, w = x.shape
    grid_spec = pltpu.PrefetchScalarGridSpec(
        num_scalar_prefetch=2,
        grid=(c, t // tchunk),
        in_specs=[
            pl.BlockSpec((b, 1, tchunk, h, w),
                         lambda j, k, p, l: (0, j, k, 0, 0)),
        ],
        out_specs=pl.BlockSpec((b, 1, tchunk, h, w),
                               lambda j, k, p, l: (0, j, k, 0, 0)),
    )
    return pl.pallas_call(
        _mix_x_body,
        grid_spec=grid_spec,
        out_shape=jax.ShapeDtypeStruct(x.shape, jnp.float32),
        compiler_params=pltpu.CompilerParams(
            dimension_semantics=("arbitrary", "arbitrary"),
        ),
    )(partner, lam_rows, x)


def _labels_body(lab_a_ref, lab_b_ref, sub_a_ref, sub_b_ref, lam_ref,
                 lab_out_ref, sub_out_ref):
    nt = jnp.float32(_LABEL_SMOOTH / _NUM_CLASSES)
    tv = jnp.float32(1.0 - _LABEL_SMOOTH + _LABEL_SMOOTH / _NUM_CLASSES)
    lam = lam_ref[...]

    c2 = jax.lax.broadcasted_iota(jnp.int32, (_B, _NUM_CLASSES), 1)
    oh_a = jnp.where(c2 == lab_a_ref[...], tv, nt)
    oh_b = jnp.where(c2 == lab_b_ref[...], tv, nt)
    lab_out_ref[...] = lam * oh_a + (1.0 - lam) * oh_b

    c3 = jax.lax.broadcasted_iota(jnp.int32, (_B, 8, _NUM_CLASSES), 2)
    soh_a = jnp.where(c3 == sub_a_ref[...][:, :, None], tv, nt)
    soh_b = jnp.where(c3 == sub_b_ref[...][:, :, None], tv, nt)
    sub_out_ref[...] = lam[:, :, None] * soh_a + (1.0 - lam[:, :, None]) * soh_b


def _labels_mix(lab_a, lab_b, sub_a, sub_b, lam_rows):
    return pl.pallas_call(
        _labels_body,
        out_shape=[
            jax.ShapeDtypeStruct((_B, _NUM_CLASSES), jnp.float32),
            jax.ShapeDtypeStruct((_B, 8, _NUM_CLASSES), jnp.float32),
        ],
    )(lab_a, lab_b, sub_a, sub_b, lam_rows.reshape(_B, 1))


def kernel(x_video_rgb, labels_action, labels_subclips_action):
    ts = jnp.squeeze(labels_subclips_action, axis=-1)
    mask = jnp.all(ts != _IGNORE_CLS, axis=-1)
    k = jnp.sum(mask)
    no_mix = k <= 1
    order = jnp.argsort(jnp.logical_not(mask), stable=True)
    rank = jnp.cumsum(mask) - 1
    partner = order[jnp.clip(k - 1 - rank, 0, _B - 1)].astype(jnp.int32)
    lam = jax.random.beta(jax.random.key(1), _ALPHA, _ALPHA)
    mix_on = mask & jnp.logical_not(no_mix)
    lam_rows = jnp.where(mix_on, lam, 1.0).astype(jnp.float32)

    x_out = _mix_x(x_video_rgb, partner, lam_rows, tchunk=4)

    subclips_ignore_index = labels_subclips_action == _IGNORE_CLS
    val_tmp = jnp.where(subclips_ignore_index, 0, labels_subclips_action)
    sub_self = jnp.squeeze(val_tmp, axis=-1)
    labels_out, labels_subclips_out = _labels_mix(
        labels_action, labels_action[partner],
        sub_self, sub_self[partner], lam_rows)
    return (x_out, labels_out, labels_subclips_out, subclips_ignore_index)
